# baseline (speedup 1.0000x reference)
import jax
import jax.numpy as jnp
from jax.experimental import pallas as pl


_S_BLK = 256


def _add_kernel(in_ref, emb_ref, out_ref):
    out_ref[...] = in_ref[...] + emb_ref[...][None, :, :]


def kernel(inputs, embeddings):
    batch, seq_len, dim = inputs.shape
    pos = embeddings[:seq_len]
    n_s = seq_len // _S_BLK

    return pl.pallas_call(
        _add_kernel,
        grid=(n_s,),
        in_specs=[
            pl.BlockSpec((batch, _S_BLK, dim), lambda s: (0, s, 0)),
            pl.BlockSpec((_S_BLK, dim), lambda s: (s, 0)),
        ],
        out_specs=pl.BlockSpec((batch, _S_BLK, dim), lambda s: (0, s, 0)),
        out_shape=jax.ShapeDtypeStruct(inputs.shape, inputs.dtype),
    )(inputs, pos)


# speedup vs baseline: 1.0093x; 1.0093x over previous
import jax
import jax.numpy as jnp
from jax.experimental import pallas as pl


_S_BLK = 512


def _add_kernel(in_ref, emb_ref, out_ref):
    out_ref[...] = in_ref[...] + emb_ref[...][None, :, :]


def kernel(inputs, embeddings):
    batch, seq_len, dim = inputs.shape
    pos = embeddings[:seq_len]
    n_s = seq_len // _S_BLK

    return pl.pallas_call(
        _add_kernel,
        grid=(n_s,),
        in_specs=[
            pl.BlockSpec((batch, _S_BLK, dim), lambda s: (0, s, 0)),
            pl.BlockSpec((_S_BLK, dim), lambda s: (s, 0)),
        ],
        out_specs=pl.BlockSpec((batch, _S_BLK, dim), lambda s: (0, s, 0)),
        out_shape=jax.ShapeDtypeStruct(inputs.shape, inputs.dtype),
    )(inputs, pos)


# speedup vs baseline: 1.0135x; 1.0042x over previous
import jax
import jax.numpy as jnp
from jax.experimental import pallas as pl


_S_BLK = 2048


def _add_kernel(in_ref, emb_ref, out_ref):
    out_ref[...] = in_ref[...] + emb_ref[...][None, :, :]


def kernel(inputs, embeddings):
    batch, seq_len, dim = inputs.shape
    pos = embeddings[:seq_len]
    n_s = seq_len // _S_BLK

    return pl.pallas_call(
        _add_kernel,
        grid=(n_s, batch),
        in_specs=[
            pl.BlockSpec((1, _S_BLK, dim), lambda s, b: (b, s, 0)),
            pl.BlockSpec((_S_BLK, dim), lambda s, b: (s, 0)),
        ],
        out_specs=pl.BlockSpec((1, _S_BLK, dim), lambda s, b: (b, s, 0)),
        out_shape=jax.ShapeDtypeStruct(inputs.shape, inputs.dtype),
    )(inputs, pos)
